# baseline (speedup 1.0000x reference)
import jax
import jax.numpy as jnp
from jax import lax
from jax.experimental import pallas as pl
from jax.experimental.pallas import tpu as pltpu
from jax.experimental.pallas import tpu_sc as plsc


def _add_kernel(p_ref, t_ref, o_ref):
    o_ref[...] = p_ref[...] + t_ref[...]


import jax
import jax.numpy as jnp
from jax import lax
from jax.experimental import pallas as pl
from jax.experimental.pallas import tpu as pltpu
from jax.experimental.pallas import tpu_sc as plsc

_B, _P, _D = 64, 576, 768
_NC, _NS = 2, 16
_NW = _NC * _NS
_CH = 32
_NK = _P // _CH
_NQ = 2 * _NK
_LANES = 16


def _sc_body(patches_hbm, table_hbm, out_hbm, tbuf, buf0, buf1,
             isem0, isem1, osem0, osem1):
    c = lax.axis_index("c")
    s = lax.axis_index("s")
    w = s * _NC + c
    b0 = 2 * w

    bufs = (buf0, buf1)
    isems = (isem0, isem1)
    osems = (osem0, osem1)

    def base(q):
        return pl.multiple_of((b0 + q % 2) * _P + (q // 2) * _CH, _CH)

    def in_copy(q, j):
        return pltpu.make_async_copy(
            patches_hbm.at[pl.ds(base(q), _CH)], bufs[j], isems[j])

    def out_copy(q, j):
        return pltpu.make_async_copy(
            bufs[j], out_hbm.at[pl.ds(base(q), _CH)], osems[j])

    def add_chunk(j):
        buf = bufs[j]

        def row(r, carry):
            for g in range(_D // _LANES):
                sl = pl.ds(g * _LANES, _LANES)
                buf[r, sl] = buf[r, sl] + tbuf[r, sl]
            return carry
        lax.fori_loop(0, _CH, row, 0)

    def chunk(q, j, first=False, last=False):
        jn = 1 - j
        if j == 0:
            pltpu.sync_copy(table_hbm.at[pl.ds(pl.multiple_of((q // 2) * _CH, _CH), _CH)], tbuf)
        in_copy(q, j).wait()
        if not first:
            out_copy(q - 1, jn).wait()
        if not last:
            in_copy(q + 1, jn).start()
        add_chunk(j)
        out_copy(q, j).start()

    in_copy(0, 0).start()
    chunk(0, 0, first=True)
    chunk(1, 1)

    def pair(i, carry):
        chunk(2 * i, 0)
        chunk(2 * i + 1, 1)
        return carry

    lax.fori_loop(1, _NQ // 2 - 1, pair, 0)

    chunk(_NQ - 2, 0)
    chunk(_NQ - 1, 1, last=True)
    out_copy(_NQ - 1, (_NQ - 1) % 2).wait()


def _sc_call(rows, position_table):
    mesh = plsc.VectorSubcoreMesh(core_axis_name="c", subcore_axis_name="s")
    return pl.kernel(
        _sc_body,
        out_type=jax.ShapeDtypeStruct(rows.shape, rows.dtype),
        mesh=mesh,
        scratch_types=[
            pltpu.VMEM((_CH, _D), jnp.float32),
            pltpu.VMEM((_CH, _D), jnp.float32),
            pltpu.VMEM((_CH, _D), jnp.float32),
            pltpu.SemaphoreType.DMA,
            pltpu.SemaphoreType.DMA,
            pltpu.SemaphoreType.DMA,
            pltpu.SemaphoreType.DMA,
        ],
    )(rows, position_table)


def kernel(encoded_patches, position_table):
    B, P, D = encoded_patches.shape
    BB = 8
    tc_out = pl.pallas_call(
        _add_kernel,
        grid=(B // BB,),
        in_specs=[
            pl.BlockSpec((BB, P, D), lambda i: (i, 0, 0)),
            pl.BlockSpec((P, D), lambda i: (0, 0)),
        ],
        out_specs=pl.BlockSpec((BB, P, D), lambda i: (i, 0, 0)),
        out_shape=jax.ShapeDtypeStruct((B, P, D), encoded_patches.dtype),
    )(encoded_patches, position_table)
    sc_out = _sc_call(encoded_patches.reshape(B * P, D), position_table)
    return (tc_out, sc_out)


# speedup vs baseline: 3.1078x; 3.1078x over previous
import jax
import jax.numpy as jnp
from jax.experimental import pallas as pl
from jax.experimental.pallas import tpu as pltpu


def _add_kernel(p_ref, t_ref, o_ref):
    o_ref[...] = p_ref[...] + t_ref[...]


def kernel(encoded_patches, position_table):
    B, P, D = encoded_patches.shape
    BB = 8
    return pl.pallas_call(
        _add_kernel,
        grid=(B // BB,),
        in_specs=[
            pl.BlockSpec((BB, P, D), lambda i: (i, 0, 0)),
            pl.BlockSpec((P, D), lambda i: (0, 0)),
        ],
        out_specs=pl.BlockSpec((BB, P, D), lambda i: (i, 0, 0)),
        out_shape=jax.ShapeDtypeStruct((B, P, D), encoded_patches.dtype),
        compiler_params=pltpu.CompilerParams(vmem_limit_bytes=128 * 1024 * 1024),
    )(encoded_patches, position_table)
